# baseline (speedup 1.0000x reference)
import functools

import jax
import jax.numpy as jnp
from jax import lax
from jax.experimental import pallas as pl
from jax.experimental.pallas import tpu as pltpu
from jax.experimental.pallas import tpu_sc as plsc

_N = 4194304
_NUM_FGS = 16
_NC = 2
_NS = 16
_NW = _NC * _NS
_PER_W = _N // _NW
_C = 8192
_NCHUNK = _PER_W // _C
_L = 16

_GS_SCALE = 1.6 / 420.0
_V_SCALE = 101.3 / 1000.0


_GATHER_DNUMS = lax.GatherDimensionNumbers(
    offset_dims=(), collapsed_slice_dims=(0,), start_index_map=(0,))


def _gather16(tbl, idx):
    return lax.gather(tbl, idx[:, None], _GATHER_DNUMS, slice_sizes=(1,),
                      mode=lax.GatherScatterMode.PROMISE_IN_BOUNDS)


def _chunk_compute(abuf, vbuf, fbuf, obuf, gs0_tbl, g1_tbl):
    @plsc.parallel_loop(0, _C, step=_L, unroll=8)
    def _body(i):
        s = pl.ds(i, _L)
        a = abuf[s]
        vp = vbuf[s] * _V_SCALE
        fg = fbuf[s]
        g0e = _gather16(gs0_tbl, fg)
        g1e = _gather16(g1_tbl, fg)
        ii = lax.bitcast_convert_type(vp, jnp.int32)
        seed = jnp.int32(0x5F3759DF) - lax.shift_right_logical(ii, 1)
        y = lax.bitcast_convert_type(seed, jnp.float32)
        h = vp * jnp.float32(-0.5)
        y = y * (jnp.float32(1.5) + h * y * y)
        y = y * (jnp.float32(1.5) + h * y * y)
        obuf[s] = g0e + (_GS_SCALE * a) * (jnp.float32(1.0) + g1e * y)


@functools.partial(
    pl.kernel,
    out_type=jax.ShapeDtypeStruct((_N,), jnp.float32),
    mesh=plsc.VectorSubcoreMesh(core_axis_name="c", subcore_axis_name="s"),
    scratch_types=[
        pltpu.VMEM((_NUM_FGS,), jnp.float32),
        pltpu.VMEM((_NUM_FGS,), jnp.float32),
        pltpu.VMEM((_C,), jnp.float32),
        pltpu.VMEM((_C,), jnp.float32),
        pltpu.VMEM((_C,), jnp.float32),
        pltpu.VMEM((_C,), jnp.float32),
        pltpu.VMEM((_C,), jnp.int32),
        pltpu.VMEM((_C,), jnp.int32),
        pltpu.VMEM((_C,), jnp.float32),
        pltpu.VMEM((_C,), jnp.float32),
        pltpu.SemaphoreType.DMA,
        pltpu.SemaphoreType.DMA,
        pltpu.SemaphoreType.DMA,
        pltpu.SemaphoreType.DMA,
    ],
)
def _med_sc(a_hbm, vpd_hbm, fg_hbm, gs0_hbm, g1_hbm, out_hbm,
            gs0_v, g1_v, a0, a1, v0, v1, f0, f1, o0, o1,
            sin0, sin1, sout0, sout1):
    wid = lax.axis_index("s") * _NC + lax.axis_index("c")
    base = wid * _PER_W

    pltpu.sync_copy(gs0_hbm, gs0_v)
    pltpu.sync_copy(g1_hbm, g1_v)
    gs0_tbl = gs0_v[...]
    g1_tbl = g1_v[...]

    bufs = ((a0, v0, f0, o0, sin0, sout0), (a1, v1, f1, o1, sin1, sout1))

    def start_in(j, b):
        ab, vb, fb, _, si, _ = bufs[b]
        off = base + j * _C
        return (
            pltpu.async_copy(a_hbm.at[pl.ds(off, _C)], ab, si),
            pltpu.async_copy(vpd_hbm.at[pl.ds(off, _C)], vb, si),
            pltpu.async_copy(fg_hbm.at[pl.ds(off, _C)], fb, si),
        )

    def start_out(j, b):
        _, _, _, ob, _, so = bufs[b]
        off = base + j * _C
        return pltpu.async_copy(ob, out_hbm.at[pl.ds(off, _C)], so)

    in_pend = {0: start_in(0, 0)}
    out_pend = {}
    for j in range(_NCHUNK):
        b = j & 1
        if j + 1 < _NCHUNK:
            in_pend[j + 1] = start_in(j + 1, 1 - b)
        for c in in_pend.pop(j):
            c.wait()
        if j - 2 in out_pend:
            out_pend.pop(j - 2).wait()
        ab, vb, fb, ob, _, _ = bufs[b]
        _chunk_compute(ab, vb, fb, ob, gs0_tbl, g1_tbl)
        out_pend[j] = start_out(j, b)
    for j in sorted(out_pend):
        out_pend.pop(j).wait()


def kernel(A, VPD, FGs, gs0, g1):
    return _med_sc(A, VPD, FGs, gs0, g1)


# speedup vs baseline: 1.0475x; 1.0475x over previous
import functools

import jax
import jax.numpy as jnp
from jax import lax
from jax.experimental import pallas as pl
from jax.experimental.pallas import tpu as pltpu
from jax.experimental.pallas import tpu_sc as plsc

_N = 4194304
_NUM_FGS = 16
_NC = 2
_NS = 16
_NW = _NC * _NS
_PER_W = _N // _NW
_C = 8192
_NCHUNK = _PER_W // _C
_L = 16

_GS_SCALE = 1.6 / 420.0
_V_SCALE = 101.3 / 1000.0


_GATHER_DNUMS = lax.GatherDimensionNumbers(
    offset_dims=(), collapsed_slice_dims=(0,), start_index_map=(0,))


def _gather16(tbl, idx):
    return lax.gather(tbl, idx[:, None], _GATHER_DNUMS, slice_sizes=(1,),
                      mode=lax.GatherScatterMode.PROMISE_IN_BOUNDS)


def _chunk_compute(abuf, vbuf, fbuf, obuf, gs0_tbl, g1_tbl):
    @plsc.parallel_loop(0, _C, step=_L, unroll=16)
    def _body(i):
        s = pl.ds(i, _L)
        a = abuf[s]
        v = vbuf[s]
        fg = fbuf[s]
        g0e = _gather16(gs0_tbl, fg)
        g1e = _gather16(g1_tbl, fg)
        ii = lax.bitcast_convert_type(v, jnp.int32)
        seed = jnp.int32(0x5F3759DF) - lax.shift_right_logical(ii, 1)
        y = lax.bitcast_convert_type(seed, jnp.float32)
        h = v * jnp.float32(-0.5)
        y = y * (jnp.float32(1.5) + h * y * y)
        obuf[s] = g0e + (_GS_SCALE * a) * (jnp.float32(1.0) + g1e * y)


@functools.partial(
    pl.kernel,
    out_type=jax.ShapeDtypeStruct((_N,), jnp.float32),
    mesh=plsc.VectorSubcoreMesh(core_axis_name="c", subcore_axis_name="s"),
    scratch_types=[
        pltpu.VMEM((_NUM_FGS,), jnp.float32),
        pltpu.VMEM((_NUM_FGS,), jnp.float32),
        pltpu.VMEM((_C,), jnp.float32),
        pltpu.VMEM((_C,), jnp.float32),
        pltpu.VMEM((_C,), jnp.float32),
        pltpu.VMEM((_C,), jnp.float32),
        pltpu.VMEM((_C,), jnp.int32),
        pltpu.VMEM((_C,), jnp.int32),
        pltpu.VMEM((_C,), jnp.float32),
        pltpu.VMEM((_C,), jnp.float32),
        pltpu.SemaphoreType.DMA,
        pltpu.SemaphoreType.DMA,
        pltpu.SemaphoreType.DMA,
        pltpu.SemaphoreType.DMA,
    ],
)
def _med_sc(a_hbm, vpd_hbm, fg_hbm, gs0_hbm, g1_hbm, out_hbm,
            gs0_v, g1_v, a0, a1, v0, v1, f0, f1, o0, o1,
            sin0, sin1, sout0, sout1):
    wid = lax.axis_index("s") * _NC + lax.axis_index("c")
    base = wid * _PER_W

    pltpu.sync_copy(gs0_hbm, gs0_v)
    pltpu.sync_copy(g1_hbm, g1_v)
    gs0_tbl = gs0_v[...]
    g1_tbl = g1_v[...] * jnp.float32(_V_SCALE**-0.5)

    bufs = ((a0, v0, f0, o0, sin0, sout0), (a1, v1, f1, o1, sin1, sout1))

    def start_in(j, b):
        ab, vb, fb, _, si, _ = bufs[b]
        off = base + j * _C
        return (
            pltpu.async_copy(a_hbm.at[pl.ds(off, _C)], ab, si),
            pltpu.async_copy(vpd_hbm.at[pl.ds(off, _C)], vb, si),
            pltpu.async_copy(fg_hbm.at[pl.ds(off, _C)], fb, si),
        )

    def start_out(j, b):
        _, _, _, ob, _, so = bufs[b]
        off = base + j * _C
        return pltpu.async_copy(ob, out_hbm.at[pl.ds(off, _C)], so)

    in_pend = {0: start_in(0, 0)}
    out_pend = {}
    for j in range(_NCHUNK):
        b = j & 1
        if j + 1 < _NCHUNK:
            in_pend[j + 1] = start_in(j + 1, 1 - b)
        for c in in_pend.pop(j):
            c.wait()
        if j - 2 in out_pend:
            out_pend.pop(j - 2).wait()
        ab, vb, fb, ob, _, _ = bufs[b]
        _chunk_compute(ab, vb, fb, ob, gs0_tbl, g1_tbl)
        out_pend[j] = start_out(j, b)
    for j in sorted(out_pend):
        out_pend.pop(j).wait()


def kernel(A, VPD, FGs, gs0, g1):
    return _med_sc(A, VPD, FGs, gs0, g1)


# speedup vs baseline: 1.1050x; 1.0549x over previous
import functools

import jax
import jax.numpy as jnp
from jax import lax
from jax.experimental import pallas as pl
from jax.experimental.pallas import tpu as pltpu
from jax.experimental.pallas import tpu_sc as plsc

_N = 4194304
_NUM_FGS = 16
_NC = 2
_NS = 16
_NW = _NC * _NS
_PER_W = _N // _NW
_C = 8192
_NCHUNK = _PER_W // _C
_L = 16

_GS_SCALE = 1.6 / 420.0
_V_SCALE = 101.3 / 1000.0


_GATHER_DNUMS = lax.GatherDimensionNumbers(
    offset_dims=(), collapsed_slice_dims=(0,), start_index_map=(0,))


def _gather16(tbl, idx):
    return lax.gather(tbl, idx[:, None], _GATHER_DNUMS, slice_sizes=(1,),
                      mode=lax.GatherScatterMode.PROMISE_IN_BOUNDS)


def _chunk_compute(abuf, vbuf, fbuf, obuf, gs0_tbl, g1_tbl):
    @plsc.parallel_loop(0, _C, step=_L, unroll=16)
    def _body(i):
        s = pl.ds(i, _L)
        a = abuf[s]
        v = vbuf[s]
        fg = fbuf[s]
        g0e = _gather16(gs0_tbl, fg)
        g1e = _gather16(g1_tbl, fg)
        ii = lax.bitcast_convert_type(v, jnp.int32)
        seed = jnp.int32(0x5F3759DF) - lax.shift_right_logical(ii, 1)
        y = lax.bitcast_convert_type(seed, jnp.float32)
        h = v * jnp.float32(-0.5)
        y = y * (jnp.float32(1.5) + h * y * y)
        obuf[s] = g0e + (_GS_SCALE * a) * (jnp.float32(1.0) + g1e * y)


@functools.partial(
    pl.kernel,
    out_type=jax.ShapeDtypeStruct((_N,), jnp.float32),
    mesh=plsc.VectorSubcoreMesh(core_axis_name="c", subcore_axis_name="s"),
    scratch_types=[
        pltpu.VMEM((_NUM_FGS,), jnp.float32),
        pltpu.VMEM((_NUM_FGS,), jnp.float32),
        pltpu.VMEM((_C,), jnp.float32),
        pltpu.VMEM((_C,), jnp.float32),
        pltpu.VMEM((_C,), jnp.float32),
        pltpu.VMEM((_C,), jnp.float32),
        pltpu.VMEM((_C,), jnp.int32),
        pltpu.VMEM((_C,), jnp.int32),
        pltpu.VMEM((_C,), jnp.float32),
        pltpu.VMEM((_C,), jnp.float32),
        pltpu.SemaphoreType.DMA,
        pltpu.SemaphoreType.DMA,
        pltpu.SemaphoreType.DMA,
        pltpu.SemaphoreType.DMA,
    ],
)
def _med_sc(a_hbm, vpd_hbm, fg_hbm, gs0_hbm, g1_hbm, out_hbm,
            gs0_v, g1_v, a0, a1, v0, v1, f0, f1, o0, o1,
            sin0, sin1, sout0, sout1):
    wid = lax.axis_index("s") * _NC + lax.axis_index("c")
    base = wid * _PER_W

    pltpu.sync_copy(gs0_hbm, gs0_v)
    pltpu.sync_copy(g1_hbm, g1_v)
    gs0_tbl = gs0_v[...]
    g1_tbl = g1_v[...] * jnp.float32(_V_SCALE**-0.5)

    bufs = ((a0, v0, f0, o0, sin0, sout0), (a1, v1, f1, o1, sin1, sout1))

    def start_in(j, b):
        ab, vb, fb, _, si, _ = bufs[b]
        off = base + j * _C
        return (
            pltpu.async_copy(a_hbm.at[pl.ds(off, _C)], ab, si),
            pltpu.async_copy(vpd_hbm.at[pl.ds(off, _C)], vb, si),
            pltpu.async_copy(fg_hbm.at[pl.ds(off, _C)], fb, si),
        )

    def start_out(j, b):
        _, _, _, ob, _, so = bufs[b]
        off = base + j * _C
        return pltpu.async_copy(ob, out_hbm.at[pl.ds(off, _C)], so)

    in_pend = {0: start_in(0, 0)}
    out_pend = {}
    for c in in_pend.pop(0):
        c.wait()
    for j in range(_NCHUNK):
        b = j & 1
        if j - 2 in out_pend:
            out_pend.pop(j - 2).wait()
        ab, vb, fb, ob, _, _ = bufs[b]
        _chunk_compute(ab, vb, fb, ob, gs0_tbl, g1_tbl)
        out_pend[j] = start_out(j, b)
    for j in sorted(out_pend):
        out_pend.pop(j).wait()


def kernel(A, VPD, FGs, gs0, g1):
    return _med_sc(A, VPD, FGs, gs0, g1)


# speedup vs baseline: 1.1601x; 1.0499x over previous
import functools

import jax
import jax.numpy as jnp
from jax import lax
from jax.experimental import pallas as pl
from jax.experimental.pallas import tpu as pltpu
from jax.experimental.pallas import tpu_sc as plsc

_N = 4194304
_NUM_FGS = 16
_NC = 2
_NS = 16
_NW = _NC * _NS
_PER_W = _N // _NW
_C = 8192
_NCHUNK = _PER_W // _C
_L = 16

_GS_SCALE = 1.6 / 420.0
_V_SCALE = 101.3 / 1000.0


_GATHER_DNUMS = lax.GatherDimensionNumbers(
    offset_dims=(), collapsed_slice_dims=(0,), start_index_map=(0,))


def _gather16(tbl, idx):
    return lax.gather(tbl, idx[:, None], _GATHER_DNUMS, slice_sizes=(1,),
                      mode=lax.GatherScatterMode.PROMISE_IN_BOUNDS)


def _chunk_compute(abuf, vbuf, fbuf, obuf, gs0_tbl, g1_tbl):
    @plsc.parallel_loop(0, _C, step=_L, unroll=2)
    def _body(i):
        s = pl.ds(i, _L)
        a = abuf[s]
        v = vbuf[s]
        fg = fbuf[s]
        g0e = _gather16(gs0_tbl, fg)
        g1e = _gather16(g1_tbl, fg)
        ii = lax.bitcast_convert_type(v, jnp.int32)
        seed = jnp.int32(0x5F3759DF) - lax.shift_right_logical(ii, 1)
        y = lax.bitcast_convert_type(seed, jnp.float32)
        h = v * jnp.float32(-0.5)
        y = y * (jnp.float32(1.5) + h * y * y)
        obuf[s] = g0e + (_GS_SCALE * a) * (jnp.float32(1.0) + g1e * y)


@functools.partial(
    pl.kernel,
    out_type=jax.ShapeDtypeStruct((_N,), jnp.float32),
    mesh=plsc.VectorSubcoreMesh(core_axis_name="c", subcore_axis_name="s"),
    scratch_types=[
        pltpu.VMEM((_NUM_FGS,), jnp.float32),
        pltpu.VMEM((_NUM_FGS,), jnp.float32),
        pltpu.VMEM((_C,), jnp.float32),
        pltpu.VMEM((_C,), jnp.float32),
        pltpu.VMEM((_C,), jnp.float32),
        pltpu.VMEM((_C,), jnp.float32),
        pltpu.VMEM((_C,), jnp.int32),
        pltpu.VMEM((_C,), jnp.int32),
        pltpu.VMEM((_C,), jnp.float32),
        pltpu.VMEM((_C,), jnp.float32),
        pltpu.SemaphoreType.DMA,
        pltpu.SemaphoreType.DMA,
        pltpu.SemaphoreType.DMA,
        pltpu.SemaphoreType.DMA,
    ],
)
def _med_sc(a_hbm, vpd_hbm, fg_hbm, gs0_hbm, g1_hbm, out_hbm,
            gs0_v, g1_v, a0, a1, v0, v1, f0, f1, o0, o1,
            sin0, sin1, sout0, sout1):
    wid = lax.axis_index("s") * _NC + lax.axis_index("c")
    base = wid * _PER_W

    pltpu.sync_copy(gs0_hbm, gs0_v)
    pltpu.sync_copy(g1_hbm, g1_v)
    gs0_tbl = gs0_v[...]
    g1_tbl = g1_v[...] * jnp.float32(_V_SCALE**-0.5)

    bufs = ((a0, v0, f0, o0, sin0, sout0), (a1, v1, f1, o1, sin1, sout1))

    def start_in(j, b):
        ab, vb, fb, _, si, _ = bufs[b]
        off = base + j * _C
        return (
            pltpu.async_copy(a_hbm.at[pl.ds(off, _C)], ab, si),
            pltpu.async_copy(vpd_hbm.at[pl.ds(off, _C)], vb, si),
            pltpu.async_copy(fg_hbm.at[pl.ds(off, _C)], fb, si),
        )

    def start_out(j, b):
        _, _, _, ob, _, so = bufs[b]
        off = base + j * _C
        return pltpu.async_copy(ob, out_hbm.at[pl.ds(off, _C)], so)

    in_pend = {0: start_in(0, 0)}
    out_pend = {}
    for j in range(_NCHUNK):
        b = j & 1
        if j + 1 < _NCHUNK:
            in_pend[j + 1] = start_in(j + 1, 1 - b)
        for c in in_pend.pop(j):
            c.wait()
        if j - 2 in out_pend:
            out_pend.pop(j - 2).wait()
        ab, vb, fb, ob, _, _ = bufs[b]
        _chunk_compute(ab, vb, fb, ob, gs0_tbl, g1_tbl)
        out_pend[j] = start_out(j, b)
    for j in sorted(out_pend):
        out_pend.pop(j).wait()


def kernel(A, VPD, FGs, gs0, g1):
    return _med_sc(A, VPD, FGs, gs0, g1)


# speedup vs baseline: 1.2793x; 1.1027x over previous
import functools

import jax
import jax.numpy as jnp
from jax import lax
from jax.experimental import pallas as pl
from jax.experimental.pallas import tpu as pltpu
from jax.experimental.pallas import tpu_sc as plsc

_N = 4194304
_NUM_FGS = 16
_NC = 2
_NS = 16
_NW = _NC * _NS
_PER_W = _N // _NW
_C = 8192
_NCHUNK = _PER_W // _C
_NBUF_IN = 3
_NBUF_OUT = 2
_L = 16

_GS_SCALE = 1.6 / 420.0
_V_SCALE = 101.3 / 1000.0


_GATHER_DNUMS = lax.GatherDimensionNumbers(
    offset_dims=(), collapsed_slice_dims=(0,), start_index_map=(0,))


def _gather16(tbl, idx):
    return lax.gather(tbl, idx[:, None], _GATHER_DNUMS, slice_sizes=(1,),
                      mode=lax.GatherScatterMode.PROMISE_IN_BOUNDS)


def _chunk_compute(abuf, vbuf, fbuf, obuf, gs0_tbl, g1_tbl):
    @plsc.parallel_loop(0, _C, step=_L, unroll=4)
    def _body(i):
        s = pl.ds(i, _L)
        a = abuf[s]
        v = vbuf[s]
        fg = fbuf[s]
        g0e = _gather16(gs0_tbl, fg)
        g1e = _gather16(g1_tbl, fg)
        ii = lax.bitcast_convert_type(v, jnp.int32)
        seed = jnp.int32(0x5F3759DF) - lax.shift_right_logical(ii, 1)
        y = lax.bitcast_convert_type(seed, jnp.float32)
        h = v * jnp.float32(-0.5)
        y = y * (jnp.float32(1.5) + h * y * y)
        obuf[s] = g0e + (_GS_SCALE * a) * (jnp.float32(1.0) + g1e * y)


@functools.partial(
    pl.kernel,
    out_type=jax.ShapeDtypeStruct((_N,), jnp.float32),
    mesh=plsc.VectorSubcoreMesh(core_axis_name="c", subcore_axis_name="s"),
    scratch_types=(
        [pltpu.VMEM((_NUM_FGS,), jnp.float32)] * 2
        + [pltpu.VMEM((_C,), jnp.float32)] * _NBUF_IN
        + [pltpu.VMEM((_C,), jnp.float32)] * _NBUF_IN
        + [pltpu.VMEM((_C,), jnp.int32)] * _NBUF_IN
        + [pltpu.VMEM((_C,), jnp.float32)] * _NBUF_OUT
        + [pltpu.SemaphoreType.DMA] * (_NBUF_IN + _NBUF_OUT)
    ),
)
def _med_sc(a_hbm, vpd_hbm, fg_hbm, gs0_hbm, g1_hbm, out_hbm, *refs):
    gs0_v, g1_v = refs[0], refs[1]
    abufs = refs[2:2 + _NBUF_IN]
    vbufs = refs[2 + _NBUF_IN:2 + 2 * _NBUF_IN]
    fbufs = refs[2 + 2 * _NBUF_IN:2 + 3 * _NBUF_IN]
    obufs = refs[2 + 3 * _NBUF_IN:2 + 3 * _NBUF_IN + _NBUF_OUT]
    sems = refs[2 + 3 * _NBUF_IN + _NBUF_OUT:]
    sin = sems[:_NBUF_IN]
    sout = sems[_NBUF_IN:]

    wid = lax.axis_index("s") * _NC + lax.axis_index("c")
    base = wid * _PER_W

    def start_in(j):
        b = j % _NBUF_IN
        off = base + j * _C
        return (
            pltpu.async_copy(a_hbm.at[pl.ds(off, _C)], abufs[b], sin[b]),
            pltpu.async_copy(vpd_hbm.at[pl.ds(off, _C)], vbufs[b], sin[b]),
            pltpu.async_copy(fg_hbm.at[pl.ds(off, _C)], fbufs[b], sin[b]),
        )

    def start_out(j):
        b = j % _NBUF_OUT
        off = base + j * _C
        return pltpu.async_copy(obufs[b], out_hbm.at[pl.ds(off, _C)], sout[b])

    in_pend = {j: start_in(j) for j in range(_NBUF_IN - 1)}

    pltpu.sync_copy(gs0_hbm, gs0_v)
    pltpu.sync_copy(g1_hbm, g1_v)
    gs0_tbl = gs0_v[...]
    g1_tbl = g1_v[...] * jnp.float32(_V_SCALE**-0.5)

    out_pend = {}
    for j in range(_NCHUNK):
        nxt = j + _NBUF_IN - 1
        if nxt < _NCHUNK:
            in_pend[nxt] = start_in(nxt)
        for c in in_pend.pop(j):
            c.wait()
        if j - _NBUF_OUT in out_pend:
            out_pend.pop(j - _NBUF_OUT).wait()
        b = j % _NBUF_IN
        _chunk_compute(abufs[b], vbufs[b], fbufs[b], obufs[j % _NBUF_OUT],
                       gs0_tbl, g1_tbl)
        out_pend[j] = start_out(j)
    for j in sorted(out_pend):
        out_pend.pop(j).wait()


def kernel(A, VPD, FGs, gs0, g1):
    return _med_sc(A, VPD, FGs, gs0, g1)
